# baseline (speedup 1.0000x reference)
import numpy as np
import jax
import jax.numpy as jnp
from jax.experimental import pallas as pl
from jax.experimental.pallas import tpu as pltpu

_NL = 4
_R = 7
_IMG = 512
_HW = (256, 128, 64, 32)
_T_BIG = 64
_T_SMALL = 32


def _pe_table_np(length, dim):
    pos = np.arange(length, dtype=np.float64)[:, None]
    idx = np.arange(dim)
    div = np.power(10000.0, (2.0 * (idx // 2)) / float(dim))[None, :]
    angle = pos / div
    return np.where((idx[None, :] % 2) == 0, np.sin(angle), np.cos(angle)).astype(np.float32)


def _roi_kernel(s_ref, bnd_ref, conv0, conv1, conv2, conv3, pe_ref,
                pool_out, pe_out, s_big, s_mid, s_small, sem):
    g = pl.program_id(0)
    lvl = s_ref[g, 0]
    b = s_ref[g, 1]
    sy = pl.multiple_of(s_ref[g, 2], 8)
    sx = pl.multiple_of(s_ref[g, 3], 128)
    bnd = bnd_ref[0]
    pr0 = bnd[0]
    pr1 = bnd[1]
    pc0 = bnd[2]
    pc1 = bnd[3]
    er0 = bnd[4]
    er1 = bnd[5]
    ec0 = bnd[6]
    ec1 = bnd[7]

    @pl.when(lvl == 0)
    def _():
        c = pltpu.make_async_copy(
            conv0.at[b, :, pl.ds(sy, 64), pl.ds(sx, 128)], s_big, sem)
        c.start()
        c.wait()

    @pl.when(lvl == 1)
    def _():
        c = pltpu.make_async_copy(
            conv1.at[b, :, pl.ds(sy, 64), pl.ds(sx, 128)], s_big, sem)
        c.start()
        c.wait()

    @pl.when(lvl == 2)
    def _():
        c = pltpu.make_async_copy(
            conv2.at[b, :, pl.ds(sy, 40), pl.ds(sx, 64)], s_mid, sem)
        c.start()
        c.wait()

    @pl.when(lvl == 3)
    def _():
        c = pltpu.make_async_copy(
            conv3.at[b, :, pl.ds(sy, 32), pl.ds(sx, 32)], s_small, sem)
        c.start()
        c.wait()

    cnt = ((pr1 - pr0)[:, None] * (pc1 - pc0)[None, :]).astype(jnp.float32)

    def pool(buf, th, tw):
        ch = buf.shape[0]
        hi = jax.lax.broadcasted_iota(jnp.int32, (_R, th), 1)
        mr = ((hi >= pr0[:, None]) & (hi < pr1[:, None])).astype(jnp.float32)
        wi = jax.lax.broadcasted_iota(jnp.int32, (tw, _R), 0)
        mc = ((wi >= pc0[None, :]) & (wi < pc1[None, :])).astype(jnp.float32)
        a = jnp.dot(buf.reshape(ch * th, tw), mc,
                    preferred_element_type=jnp.float32)
        a = a.reshape(ch, th, _R)
        o = jax.lax.dot_general(mr, a, (((1,), (1,)), ((), ())),
                                preferred_element_type=jnp.float32)
        return jnp.transpose(o, (1, 0, 2)) / cnt[None, :, :]

    @pl.when(lvl <= 1)
    def _():
        pool_out[0] = pool(s_big[...], 64, 128)

    @pl.when(lvl == 2)
    def _():
        pool_out[0] = pool(s_mid[...], 40, 64)

    @pl.when(lvl == 3)
    def _():
        pool_out[0] = pool(s_small[...], 32, 32)

    pe = pe_ref[...]
    d = pe.shape[1] // 2
    pi = jax.lax.broadcasted_iota(jnp.int32, (_R, _IMG), 1)
    mrow = ((pi >= er0[:, None]) & (pi < er1[:, None])).astype(jnp.float32)
    mcol = ((pi >= ec0[:, None]) & (pi < ec1[:, None])).astype(jnp.float32)
    ff = jnp.dot(mrow, pe[:, :d], preferred_element_type=jnp.float32)
    ft = jnp.dot(mcol, pe[:, d:], preferred_element_type=jnp.float32)
    hl = (er1 - er0).astype(jnp.float32)
    wl = (ec1 - ec0).astype(jnp.float32)
    pf = jnp.transpose(ff / hl[:, None], (1, 0))
    pt = jnp.transpose(ft / wl[:, None], (1, 0))
    pe_out[0] = jnp.concatenate(
        [jnp.broadcast_to(pf[:, :, None], (d, _R, _R)),
         jnp.broadcast_to(pt[:, None, :], (d, _R, _R))], axis=0)


def _bin_bounds(length, out):
    idx = jnp.arange(out, dtype=jnp.int32)
    starts = (idx * length[..., None]) // out
    ends = -((-(idx + 1) * length[..., None]) // out)
    return starts, ends


def kernel(rois, conv_out_0, conv_out_1, conv_out_2, conv_out_3):
    conv = [conv_out_0, conv_out_1, conv_out_2, conv_out_3]
    B, N = rois.shape[0], rois.shape[1]
    C = conv_out_0.shape[1]
    G = B * N
    heights = jnp.asarray([int(f.shape[-2]) for f in conv], jnp.int32)
    widths = jnp.asarray([int(f.shape[-1]) for f in conv], jnp.int32)

    size = jnp.sqrt((rois[..., 2] - rois[..., 0]) * (rois[..., 3] - rois[..., 1]))
    lvl = jnp.clip(jnp.trunc(jnp.log(size * 0.1) / np.log(2.0)).astype(jnp.int32),
                   0, _NL - 1)
    stride_vals = jnp.asarray([2.0 ** (i + 1) for i in range(_NL)], jnp.float32)
    stride = stride_vals[lvl]
    x1 = jnp.rint(rois[..., 0] / stride).astype(jnp.int32)
    y1 = jnp.rint(rois[..., 1] / stride).astype(jnp.int32)
    x2 = jnp.rint(rois[..., 2] / stride).astype(jnp.int32)
    y2 = jnp.rint(rois[..., 3] / stride).astype(jnp.int32)
    height = heights[lvl]
    width = widths[lvl]
    xx1, xx2, yy1 = x1, x2, y1
    yy2 = jnp.minimum(y2, height - 1)
    for _ in range(_R):
        gy = (yy2 - yy1 + 1) < _R
        yy1 = jnp.where(gy, jnp.maximum(0, yy1 - 1), yy1)
        yy2 = jnp.where(gy, jnp.minimum(height - 1, yy2 + 1), yy2)
        gx = (xx2 - xx1 + 1) < _R
        xx1 = jnp.where(gx, jnp.maximum(0, xx1 - 1), xx1)
        xx2 = jnp.where(gx, jnp.minimum(width - 1, xx2 + 1), xx2)

    hlen = jnp.minimum(yy2, height - 1) - yy1 + 1
    wlen = jnp.minimum(xx2, width - 1) - xx1 + 1
    trows = jnp.asarray([64, 64, 40, 32], jnp.int32)[lvl]
    sy = jnp.minimum((yy1 // 8) * 8, height - trows)
    sx = jnp.where(lvl == 0,
                   jnp.minimum((xx1 // 128) * 128, width - 128),
                   0)

    hs, he = _bin_bounds(hlen, _R)
    ws, we = _bin_bounds(wlen, _R)
    pr0 = (yy1 - sy)[..., None] + hs
    pr1 = (yy1 - sy)[..., None] + he
    pc0 = (xx1 - sx)[..., None] + ws
    pc1 = (xx1 - sx)[..., None] + we

    s_int = stride.astype(jnp.int32)
    r0 = s_int * yy1
    lf = jnp.minimum(s_int * yy2, _IMG) - r0
    lt = jnp.minimum(s_int * (xx2 - xx1), _IMG)
    ehs, ehe = _bin_bounds(lf, _R)
    ews, ewe = _bin_bounds(lt, _R)
    er0 = r0[..., None] + ehs
    er1 = r0[..., None] + ehe

    bounds = jnp.stack([pr0, pr1, pc0, pc1, er0, er1, ews, ewe],
                       axis=-2).reshape(G, 8, _R).astype(jnp.int32)
    b_idx = jnp.broadcast_to(jnp.arange(B, dtype=jnp.int32)[:, None], (B, N))
    scalars = jnp.stack([lvl, b_idx, sy, sx], axis=-1).reshape(G, 4).astype(jnp.int32)

    pe_np = np.concatenate([_pe_table_np(_IMG, C // 2),
                            _pe_table_np(_IMG, C // 2)], axis=1)
    pe_const = jnp.asarray(pe_np)

    grid_spec = pltpu.PrefetchScalarGridSpec(
        num_scalar_prefetch=1,
        grid=(G,),
        in_specs=[
            pl.BlockSpec((1, 8, _R), lambda g, s: (g, 0, 0)),
            pl.BlockSpec(memory_space=pl.ANY),
            pl.BlockSpec(memory_space=pl.ANY),
            pl.BlockSpec(memory_space=pl.ANY),
            pl.BlockSpec(memory_space=pl.ANY),
            pl.BlockSpec((_IMG, C), lambda g, s: (0, 0)),
        ],
        out_specs=[
            pl.BlockSpec((1, C, _R, _R), lambda g, s: (g, 0, 0, 0)),
            pl.BlockSpec((1, C, _R, _R), lambda g, s: (g, 0, 0, 0)),
        ],
        scratch_shapes=[
            pltpu.VMEM((C, 64, 128), jnp.float32),
            pltpu.VMEM((C, 40, 64), jnp.float32),
            pltpu.VMEM((C, 32, 32), jnp.float32),
            pltpu.SemaphoreType.DMA,
        ],
    )
    pool_flat, pe_flat = pl.pallas_call(
        _roi_kernel,
        grid_spec=grid_spec,
        out_shape=[jax.ShapeDtypeStruct((G, C, _R, _R), jnp.float32),
                   jax.ShapeDtypeStruct((G, C, _R, _R), jnp.float32)],
        compiler_params=pltpu.CompilerParams(
            dimension_semantics=("arbitrary",)),
    )(scalars, bounds, conv_out_0, conv_out_1, conv_out_2, conv_out_3,
      pe_const)

    return (pool_flat.reshape(B, N, C, _R, _R),
            pe_flat.reshape(B, N, C, _R, _R),
            lvl.astype(jnp.int_))


# speedup vs baseline: 1.0278x; 1.0278x over previous
import numpy as np
import jax
import jax.numpy as jnp
from jax.experimental import pallas as pl
from jax.experimental.pallas import tpu as pltpu

_NL = 4
_R = 7
_IMG = 512
_HW = (256, 128, 64, 32)
_T_BIG = 64
_T_SMALL = 32


def _pe_table_np(length, dim):
    pos = np.arange(length, dtype=np.float64)[:, None]
    idx = np.arange(dim)
    div = np.power(10000.0, (2.0 * (idx // 2)) / float(dim))[None, :]
    angle = pos / div
    return np.where((idx[None, :] % 2) == 0, np.sin(angle), np.cos(angle)).astype(np.float32)


def _roi_kernel(s_ref, bnd_ref, conv0, conv1, conv2, conv3, pe_ref,
                pool_out, pe_out, s_big, s_mid, s_small, sem):
    g = pl.program_id(0)
    lvl = s_ref[g, 0]
    b = s_ref[g, 1]
    sy = pl.multiple_of(s_ref[g, 2], 8)
    sx = pl.multiple_of(s_ref[g, 3], 128)
    bnd = bnd_ref[0]
    pr0 = bnd[0]
    pr1 = bnd[1]
    pc0 = bnd[2]
    pc1 = bnd[3]
    er0 = bnd[4]
    er1 = bnd[5]
    ec0 = bnd[6]
    ec1 = bnd[7]

    @pl.when(lvl == 0)
    def _():
        c = pltpu.make_async_copy(
            conv0.at[b, :, pl.ds(sy, 64), pl.ds(sx, 128)], s_big, sem)
        c.start()
        c.wait()

    @pl.when(lvl == 1)
    def _():
        c = pltpu.make_async_copy(
            conv1.at[b, :, pl.ds(sy, 64), pl.ds(sx, 128)], s_big, sem)
        c.start()
        c.wait()

    @pl.when(lvl == 2)
    def _():
        c = pltpu.make_async_copy(
            conv2.at[b, :, pl.ds(sy, 40), pl.ds(sx, 64)], s_mid, sem)
        c.start()
        c.wait()

    @pl.when(lvl == 3)
    def _():
        c = pltpu.make_async_copy(
            conv3.at[b, :, pl.ds(sy, 32), pl.ds(sx, 32)], s_small, sem)
        c.start()
        c.wait()

    cnt = ((pr1 - pr0)[:, None] * (pc1 - pc0)[None, :]).astype(jnp.float32)

    def pool(buf, th, tw):
        ch = buf.shape[0]
        hi = jax.lax.broadcasted_iota(jnp.int32, (_R, th), 1)
        mr = ((hi >= pr0[:, None]) & (hi < pr1[:, None])).astype(jnp.float32)
        wi = jax.lax.broadcasted_iota(jnp.int32, (tw, _R), 0)
        mc = ((wi >= pc0[None, :]) & (wi < pc1[None, :])).astype(jnp.float32)
        a = jax.lax.dot_general(mr, buf, (((1,), (1,)), ((), ())),
                                preferred_element_type=jnp.float32)
        o = jnp.dot(a.reshape(_R * ch, tw), mc,
                    preferred_element_type=jnp.float32)
        o = o.reshape(_R, ch, _R)
        return jnp.transpose(o, (1, 0, 2)) / cnt[None, :, :]

    @pl.when(lvl <= 1)
    def _():
        pool_out[0] = pool(s_big[...], 64, 128)

    @pl.when(lvl == 2)
    def _():
        pool_out[0] = pool(s_mid[...], 40, 64)

    @pl.when(lvl == 3)
    def _():
        pool_out[0] = pool(s_small[...], 32, 32)

    pe = pe_ref[...]
    d = pe.shape[1] // 2
    pi = jax.lax.broadcasted_iota(jnp.int32, (_R, _IMG), 1)
    mrow = ((pi >= er0[:, None]) & (pi < er1[:, None])).astype(jnp.float32)
    mcol = ((pi >= ec0[:, None]) & (pi < ec1[:, None])).astype(jnp.float32)
    ff = jnp.dot(mrow, pe[:, :d], preferred_element_type=jnp.float32)
    ft = jnp.dot(mcol, pe[:, d:], preferred_element_type=jnp.float32)
    hl = (er1 - er0).astype(jnp.float32)
    wl = (ec1 - ec0).astype(jnp.float32)
    pf = jnp.transpose(ff / hl[:, None], (1, 0))
    pt = jnp.transpose(ft / wl[:, None], (1, 0))
    pe_out[0] = jnp.concatenate(
        [jnp.broadcast_to(pf[:, :, None], (d, _R, _R)),
         jnp.broadcast_to(pt[:, None, :], (d, _R, _R))], axis=0)


def _bin_bounds(length, out):
    idx = jnp.arange(out, dtype=jnp.int32)
    starts = (idx * length[..., None]) // out
    ends = -((-(idx + 1) * length[..., None]) // out)
    return starts, ends


def kernel(rois, conv_out_0, conv_out_1, conv_out_2, conv_out_3):
    conv = [conv_out_0, conv_out_1, conv_out_2, conv_out_3]
    B, N = rois.shape[0], rois.shape[1]
    C = conv_out_0.shape[1]
    G = B * N
    heights = jnp.asarray([int(f.shape[-2]) for f in conv], jnp.int32)
    widths = jnp.asarray([int(f.shape[-1]) for f in conv], jnp.int32)

    size = jnp.sqrt((rois[..., 2] - rois[..., 0]) * (rois[..., 3] - rois[..., 1]))
    lvl = jnp.clip(jnp.trunc(jnp.log(size * 0.1) / np.log(2.0)).astype(jnp.int32),
                   0, _NL - 1)
    stride_vals = jnp.asarray([2.0 ** (i + 1) for i in range(_NL)], jnp.float32)
    stride = stride_vals[lvl]
    x1 = jnp.rint(rois[..., 0] / stride).astype(jnp.int32)
    y1 = jnp.rint(rois[..., 1] / stride).astype(jnp.int32)
    x2 = jnp.rint(rois[..., 2] / stride).astype(jnp.int32)
    y2 = jnp.rint(rois[..., 3] / stride).astype(jnp.int32)
    height = heights[lvl]
    width = widths[lvl]
    xx1, xx2, yy1 = x1, x2, y1
    yy2 = jnp.minimum(y2, height - 1)
    for _ in range(_R):
        gy = (yy2 - yy1 + 1) < _R
        yy1 = jnp.where(gy, jnp.maximum(0, yy1 - 1), yy1)
        yy2 = jnp.where(gy, jnp.minimum(height - 1, yy2 + 1), yy2)
        gx = (xx2 - xx1 + 1) < _R
        xx1 = jnp.where(gx, jnp.maximum(0, xx1 - 1), xx1)
        xx2 = jnp.where(gx, jnp.minimum(width - 1, xx2 + 1), xx2)

    hlen = jnp.minimum(yy2, height - 1) - yy1 + 1
    wlen = jnp.minimum(xx2, width - 1) - xx1 + 1
    trows = jnp.asarray([64, 64, 40, 32], jnp.int32)[lvl]
    sy = jnp.minimum((yy1 // 8) * 8, height - trows)
    sx = jnp.where(lvl == 0,
                   jnp.minimum((xx1 // 128) * 128, width - 128),
                   0)

    hs, he = _bin_bounds(hlen, _R)
    ws, we = _bin_bounds(wlen, _R)
    pr0 = (yy1 - sy)[..., None] + hs
    pr1 = (yy1 - sy)[..., None] + he
    pc0 = (xx1 - sx)[..., None] + ws
    pc1 = (xx1 - sx)[..., None] + we

    s_int = stride.astype(jnp.int32)
    r0 = s_int * yy1
    lf = jnp.minimum(s_int * yy2, _IMG) - r0
    lt = jnp.minimum(s_int * (xx2 - xx1), _IMG)
    ehs, ehe = _bin_bounds(lf, _R)
    ews, ewe = _bin_bounds(lt, _R)
    er0 = r0[..., None] + ehs
    er1 = r0[..., None] + ehe

    bounds = jnp.stack([pr0, pr1, pc0, pc1, er0, er1, ews, ewe],
                       axis=-2).reshape(G, 8, _R).astype(jnp.int32)
    b_idx = jnp.broadcast_to(jnp.arange(B, dtype=jnp.int32)[:, None], (B, N))
    scalars = jnp.stack([lvl, b_idx, sy, sx], axis=-1).reshape(G, 4).astype(jnp.int32)

    pe_np = np.concatenate([_pe_table_np(_IMG, C // 2),
                            _pe_table_np(_IMG, C // 2)], axis=1)
    pe_const = jnp.asarray(pe_np)

    grid_spec = pltpu.PrefetchScalarGridSpec(
        num_scalar_prefetch=1,
        grid=(G,),
        in_specs=[
            pl.BlockSpec((1, 8, _R), lambda g, s: (g, 0, 0)),
            pl.BlockSpec(memory_space=pl.ANY),
            pl.BlockSpec(memory_space=pl.ANY),
            pl.BlockSpec(memory_space=pl.ANY),
            pl.BlockSpec(memory_space=pl.ANY),
            pl.BlockSpec((_IMG, C), lambda g, s: (0, 0)),
        ],
        out_specs=[
            pl.BlockSpec((1, C, _R, _R), lambda g, s: (g, 0, 0, 0)),
            pl.BlockSpec((1, C, _R, _R), lambda g, s: (g, 0, 0, 0)),
        ],
        scratch_shapes=[
            pltpu.VMEM((C, 64, 128), jnp.float32),
            pltpu.VMEM((C, 40, 64), jnp.float32),
            pltpu.VMEM((C, 32, 32), jnp.float32),
            pltpu.SemaphoreType.DMA,
        ],
    )
    pool_flat, pe_flat = pl.pallas_call(
        _roi_kernel,
        grid_spec=grid_spec,
        out_shape=[jax.ShapeDtypeStruct((G, C, _R, _R), jnp.float32),
                   jax.ShapeDtypeStruct((G, C, _R, _R), jnp.float32)],
        compiler_params=pltpu.CompilerParams(
            dimension_semantics=("arbitrary",)),
    )(scalars, bounds, conv_out_0, conv_out_1, conv_out_2, conv_out_3,
      pe_const)

    return (pool_flat.reshape(B, N, C, _R, _R),
            pe_flat.reshape(B, N, C, _R, _R),
            lvl.astype(jnp.int_))
